# baseline (speedup 1.0000x reference)
import jax
import jax.numpy as jnp
from jax import lax
from jax.experimental import pallas as pl
from jax.experimental.pallas import tpu as pltpu
from jax.experimental.pallas import tpu_sc as plsc

VOCAB = 1000000
DIM = 64
SEQ = 200
BATCH = 4096

NC, NS = 2, 16
NW = NC * NS
TOKENS = BATCH * SEQ
TOK_PER_W = TOKENS // NW
CHUNK = 128
CHUNKS_PER_W = TOK_PER_W // CHUNK
POS_EXT = SEQ + CHUNK


def _sc_kernel(x_hbm, tab_hbm, pos_hbm, out_hbm, pos_v, idx_v, rows_v, sem):
    wid = lax.axis_index("s") * NC + lax.axis_index("c")
    base = wid * TOK_PER_W

    pltpu.sync_copy(pos_hbm, pos_v)

    @pl.loop(0, CHUNKS_PER_W)
    def chunk_loop(c):
        tok = base + c * CHUNK
        poff = lax.rem(c * CHUNK, SEQ)
        pltpu.sync_copy(x_hbm.at[pl.ds(tok, CHUNK)], idx_v)
        pltpu.async_copy(tab_hbm.at[idx_v], rows_v, sem).wait()

        @pl.loop(0, CHUNK)
        def add_loop(t):
            for d in range(DIM // 16):
                sl = pl.ds(16 * d, 16)
                rows_v[t, sl] = rows_v[t, sl] + pos_v[poff + t, sl]

        pltpu.sync_copy(rows_v, out_hbm.at[pl.ds(tok, CHUNK)])


@jax.jit
def kernel(x, token_emb, pos_emb):
    x_flat = x.reshape(TOKENS).astype(jnp.int32)
    pos = pos_emb[0, :SEQ, :]
    pos_ext = jnp.concatenate([pos, pos[:CHUNK]], axis=0)

    mesh = plsc.VectorSubcoreMesh(core_axis_name="c", subcore_axis_name="s")
    run = pl.kernel(
        _sc_kernel,
        out_type=jax.ShapeDtypeStruct((TOKENS, DIM), jnp.float32),
        mesh=mesh,
        scratch_types=[
            pltpu.VMEM((POS_EXT, DIM), jnp.float32),
            pltpu.VMEM((CHUNK,), jnp.int32),
            pltpu.VMEM((CHUNK, DIM), jnp.float32),
            pltpu.SemaphoreType.DMA,
        ],
        compiler_params=pltpu.CompilerParams(use_tc_tiling_on_sc=False),
    )
    out = run(x_flat, token_emb, pos_ext)
    return out.reshape(BATCH, SEQ, DIM)


# speedup vs baseline: 1.2770x; 1.2770x over previous
import jax
import jax.numpy as jnp
from jax import lax
from jax.experimental import pallas as pl
from jax.experimental.pallas import tpu as pltpu
from jax.experimental.pallas import tpu_sc as plsc

VOCAB = 1000000
DIM = 64
SEQ = 200
BATCH = 4096

NC, NS = 2, 16
NW = NC * NS
TOKENS = BATCH * SEQ
TOK_PER_W = TOKENS // NW
CHUNK = 128
NCH = TOK_PER_W // CHUNK
POS_EXT = SEQ + CHUNK
NB = 4


def _sc_kernel(x_hbm, tab_hbm, pos_hbm, out_hbm,
               pos_v, idx_v, rows_v, gsem, osem):
    wid = lax.axis_index("s") * NC + lax.axis_index("c")
    base = wid * TOK_PER_W

    pltpu.sync_copy(pos_hbm, pos_v)
    pltpu.sync_copy(x_hbm.at[pl.ds(base, TOK_PER_W)], idx_v)

    def gather(c, b):
        pltpu.async_copy(
            tab_hbm.at[idx_v.at[pl.ds(c * CHUNK, CHUNK)]],
            rows_v.at[b], gsem.at[b])

    def copyout(c, b):
        pltpu.async_copy(
            rows_v.at[b], out_hbm.at[pl.ds(base + c * CHUNK, CHUNK)],
            osem.at[b])

    gather(0, 0)
    gather(1, 1)

    @pl.loop(0, NCH)
    def chunk_loop(c):
        b = lax.rem(c, NB)

        @pl.when(c + 2 < NCH)
        def _():
            bn = lax.rem(c + 2, NB)

            @pl.when(c >= 2)
            def _():
                pltpu.make_async_copy(
                    rows_v.at[bn],
                    out_hbm.at[pl.ds(base + (c - 2) * CHUNK, CHUNK)],
                    osem.at[bn]).wait()

            gather(c + 2, bn)

        pltpu.make_async_copy(
            tab_hbm.at[idx_v.at[pl.ds(c * CHUNK, CHUNK)]],
            rows_v.at[b], gsem.at[b]).wait()

        poff = lax.rem(c * CHUNK, SEQ)

        @pl.loop(0, CHUNK, unroll=8)
        def add_loop(t):
            for d in range(DIM // 16):
                sl = pl.ds(16 * d, 16)
                rows_v[b, t, sl] = rows_v[b, t, sl] + pos_v[poff + t, sl]

        copyout(c, b)

    for k in range(NB):
        c = NCH - NB + k
        pltpu.make_async_copy(
            rows_v.at[k], out_hbm.at[pl.ds(base + c * CHUNK, CHUNK)],
            osem.at[k]).wait()


@jax.jit
def kernel(x, token_emb, pos_emb):
    x_flat = x.reshape(TOKENS).astype(jnp.int32)
    pos = pos_emb[0, :SEQ, :]
    pos_ext = jnp.concatenate([pos, pos[:CHUNK]], axis=0)

    mesh = plsc.VectorSubcoreMesh(core_axis_name="c", subcore_axis_name="s")
    run = pl.kernel(
        _sc_kernel,
        out_type=jax.ShapeDtypeStruct((TOKENS, DIM), jnp.float32),
        mesh=mesh,
        scratch_types=[
            pltpu.VMEM((POS_EXT, DIM), jnp.float32),
            pltpu.VMEM((TOK_PER_W,), jnp.int32),
            pltpu.VMEM((NB, CHUNK, DIM), jnp.float32),
            pltpu.SemaphoreType.DMA((NB,)),
            pltpu.SemaphoreType.DMA((NB,)),
        ],
        compiler_params=pltpu.CompilerParams(use_tc_tiling_on_sc=False),
    )
    out = run(x_flat, token_emb, pos_ext)
    return out.reshape(BATCH, SEQ, DIM)
